# baseline (speedup 1.0000x reference)
import functools

import jax
import jax.numpy as jnp
from jax import lax
from jax.experimental import pallas as pl
from jax.experimental.pallas import tpu as pltpu
from jax.experimental.pallas import tpu_sc as plsc

VOCAB = 100000
EMBED = 64
N = 16384

NUM_CORES = 2
NUM_SUBCORES = 16
LANES = 16

CB = 128
NROWS = N // CB
ROWS_PER_TILE = NROWS // (NUM_CORES * NUM_SUBCORES)

CHUNK = 6256
LAST_CHUNK = VOCAB - CHUNK * (NUM_SUBCORES - 1)


def _count_body(idx_hbm, out0_hbm, out1_hbm, idx_v, ones_v, zeros_v, counts_sh,
                sem):
    cc = lax.axis_index("c")
    s = lax.axis_index("s")

    row0 = cc * (NROWS // NUM_CORES) + s * ROWS_PER_TILE
    idx_cps = [
        pltpu.async_copy(idx_hbm.at[row0 + j], idx_v.at[j], sem)
        for j in range(ROWS_PER_TILE)
    ]

    one = jnp.full((LANES,), 1.0, jnp.float32)
    for k in range(CB // LANES):
        ones_v[pl.ds(k * LANES, LANES)] = one

    zero = jnp.zeros((LANES,), jnp.float32)

    def zfill(i, carry):
        zeros_v[pl.ds(pl.multiple_of(i * LANES, LANES), LANES)] = zero
        return carry

    lax.fori_loop(0, CHUNK // LANES, zfill, 0, unroll=8)

    off = pl.multiple_of(s * CHUNK, 8)

    @pl.when(s < NUM_SUBCORES - 1)
    def _():
        pltpu.sync_copy(zeros_v, counts_sh.at[pl.ds(off, CHUNK)])

    @pl.when(s == NUM_SUBCORES - 1)
    def _():
        pltpu.sync_copy(zeros_v.at[pl.ds(0, LAST_CHUNK)],
                        counts_sh.at[pl.ds(off, LAST_CHUNK)])

    for cp in idx_cps:
        cp.wait()
    plsc.subcore_barrier()

    for j in range(ROWS_PER_TILE):
        pltpu.sync_copy(ones_v, counts_sh.at[idx_v.at[j]], add=True)

    plsc.subcore_barrier()

    @pl.when(jnp.logical_and(s < NUM_SUBCORES - 1, cc == 0))
    def _():
        pltpu.sync_copy(counts_sh.at[pl.ds(off, CHUNK)],
                        out0_hbm.at[pl.ds(off, CHUNK)])

    @pl.when(jnp.logical_and(s == NUM_SUBCORES - 1, cc == 0))
    def _():
        pltpu.sync_copy(counts_sh.at[pl.ds(off, LAST_CHUNK)],
                        out0_hbm.at[pl.ds(off, LAST_CHUNK)])

    @pl.when(jnp.logical_and(s < NUM_SUBCORES - 1, cc == 1))
    def _():
        pltpu.sync_copy(counts_sh.at[pl.ds(off, CHUNK)],
                        out1_hbm.at[pl.ds(off, CHUNK)])

    @pl.when(jnp.logical_and(s == NUM_SUBCORES - 1, cc == 1))
    def _():
        pltpu.sync_copy(counts_sh.at[pl.ds(off, LAST_CHUNK)],
                        out1_hbm.at[pl.ds(off, LAST_CHUNK)])


@functools.cache
def _count_kernel():
    return pl.kernel(
        _count_body,
        out_type=[
            jax.ShapeDtypeStruct((VOCAB,), jnp.float32),
            jax.ShapeDtypeStruct((VOCAB,), jnp.float32),
        ],
        mesh=plsc.VectorSubcoreMesh(
            core_axis_name="c", subcore_axis_name="s",
            num_cores=NUM_CORES, num_subcores=NUM_SUBCORES,
        ),
        scratch_types=[
            pltpu.VMEM((ROWS_PER_TILE, CB), jnp.int32),
            pltpu.VMEM((CB,), jnp.float32),
            pltpu.VMEM((CHUNK,), jnp.float32),
            pltpu.VMEM_SHARED((VOCAB,), jnp.float32),
            pltpu.SemaphoreType.DMA,
        ],
        compiler_params=pltpu.CompilerParams(use_tc_tiling_on_sc=False),
    )


BV = 34816
NB = (VOCAB + BV - 1) // BV


HALF = EMBED // 2


def _fused_body(c0_ref, c1_ref, et0_ref, et1_ref, wt0_ref, wt1_ref, b_ref,
                o_ref, acc_ref):
    p = pl.program_id(0)
    j = pl.program_id(1)

    @pl.when(jnp.logical_and(p == 0, j == 0))
    def _():
        acc_ref[...] = jnp.zeros_like(acc_ref)

    @pl.when(p == 0)
    def _():
        col = j * BV + lax.broadcasted_iota(jnp.int32, (1, BV), 1)
        valid = col < VOCAB
        c = jnp.where(valid, (c0_ref[...] + c1_ref[...])[None, :], 0.0)
        vmask = jnp.broadcast_to(valid, (HALF, BV))
        et0 = jnp.where(vmask, et0_ref[...], 0.0)
        et1 = jnp.where(vmask, et1_ref[...], 0.0)
        acc_ref[:, :HALF] += lax.dot_general(
            c, et0, (((1,), (1,)), ((), ())),
            preferred_element_type=jnp.float32,
        )
        acc_ref[:, HALF:] += lax.dot_general(
            c, et1, (((1,), (1,)), ((), ())),
            preferred_element_type=jnp.float32,
        )

    @pl.when(p == 1)
    def _():
        res = lax.dot_general(
            acc_ref[:, :HALF], wt0_ref[...], (((1,), (0,)), ((), ())),
            preferred_element_type=jnp.float32,
        ) + lax.dot_general(
            acc_ref[:, HALF:], wt1_ref[...], (((1,), (0,)), ((), ())),
            preferred_element_type=jnp.float32,
        )
        o_ref[...] = res[0] + b_ref[...]


def _fused(c0, c1, embT, WT, b):
    return pl.pallas_call(
        _fused_body,
        grid=(2, NB),
        in_specs=[
            pl.BlockSpec((BV,), lambda p, j: (j * (1 - p) + (NB - 1) * p,)),
            pl.BlockSpec((BV,), lambda p, j: (j * (1 - p) + (NB - 1) * p,)),
            pl.BlockSpec((HALF, BV),
                         lambda p, j: (0, j * (1 - p) + (NB - 1) * p)),
            pl.BlockSpec((HALF, BV),
                         lambda p, j: (1, j * (1 - p) + (NB - 1) * p)),
            pl.BlockSpec((HALF, BV), lambda p, j: (0, j * p)),
            pl.BlockSpec((HALF, BV), lambda p, j: (1, j * p)),
            pl.BlockSpec((BV,), lambda p, j: (j * p,)),
        ],
        out_specs=pl.BlockSpec((BV,), lambda p, j: (j * p,)),
        out_shape=jax.ShapeDtypeStruct((VOCAB,), jnp.float32),
        scratch_shapes=[pltpu.VMEM((1, EMBED), jnp.float32)],
    )(c0, c1, embT, embT, WT, WT, b)


def kernel(inputs, emb, W, b):
    idx2 = inputs.astype(jnp.int32).reshape(NROWS, CB)
    c0, c1 = _count_kernel()(idx2)
    return _fused(c0, c1, emb.T, W.T, b)


# speedup vs baseline: 1.0099x; 1.0099x over previous
import functools

import jax
import jax.numpy as jnp
from jax import lax
from jax.experimental import pallas as pl
from jax.experimental.pallas import tpu as pltpu
from jax.experimental.pallas import tpu_sc as plsc

VOCAB = 100000
EMBED = 64
N = 16384

NUM_CORES = 2
NUM_SUBCORES = 16
LANES = 16

CB = 128
NROWS = N // CB
ROWS_PER_TILE = NROWS // (NUM_CORES * NUM_SUBCORES)

CHUNK = 6256
LAST_CHUNK = VOCAB - CHUNK * (NUM_SUBCORES - 1)


def _count_body(idx_hbm, out0_hbm, out1_hbm, idx_v, ones_v, zeros_v, counts_sh,
                sem):
    cc = lax.axis_index("c")
    s = lax.axis_index("s")

    row0 = cc * (NROWS // NUM_CORES) + s * ROWS_PER_TILE
    idx_cps = [
        pltpu.async_copy(idx_hbm.at[row0 + j], idx_v.at[j], sem)
        for j in range(ROWS_PER_TILE)
    ]

    one = jnp.full((LANES,), 1.0, jnp.float32)
    for k in range(CB // LANES):
        ones_v[pl.ds(k * LANES, LANES)] = one

    zero = jnp.zeros((LANES,), jnp.float32)

    def zfill(i, carry):
        zeros_v[pl.ds(pl.multiple_of(i * LANES, LANES), LANES)] = zero
        return carry

    lax.fori_loop(0, CHUNK // LANES, zfill, 0, unroll=8)

    off = pl.multiple_of(s * CHUNK, 8)

    @pl.when(s < NUM_SUBCORES - 1)
    def _():
        pltpu.sync_copy(zeros_v, counts_sh.at[pl.ds(off, CHUNK)])

    @pl.when(s == NUM_SUBCORES - 1)
    def _():
        pltpu.sync_copy(zeros_v.at[pl.ds(0, LAST_CHUNK)],
                        counts_sh.at[pl.ds(off, LAST_CHUNK)])

    for cp in idx_cps:
        cp.wait()
    plsc.subcore_barrier()

    for j in range(ROWS_PER_TILE):
        pltpu.sync_copy(ones_v, counts_sh.at[idx_v.at[j]], add=True)

    plsc.subcore_barrier()

    @pl.when(jnp.logical_and(s < NUM_SUBCORES - 1, cc == 0))
    def _():
        pltpu.sync_copy(counts_sh.at[pl.ds(off, CHUNK)],
                        out0_hbm.at[pl.ds(off, CHUNK)])

    @pl.when(jnp.logical_and(s == NUM_SUBCORES - 1, cc == 0))
    def _():
        pltpu.sync_copy(counts_sh.at[pl.ds(off, LAST_CHUNK)],
                        out0_hbm.at[pl.ds(off, LAST_CHUNK)])

    @pl.when(jnp.logical_and(s < NUM_SUBCORES - 1, cc == 1))
    def _():
        pltpu.sync_copy(counts_sh.at[pl.ds(off, CHUNK)],
                        out1_hbm.at[pl.ds(off, CHUNK)])

    @pl.when(jnp.logical_and(s == NUM_SUBCORES - 1, cc == 1))
    def _():
        pltpu.sync_copy(counts_sh.at[pl.ds(off, LAST_CHUNK)],
                        out1_hbm.at[pl.ds(off, LAST_CHUNK)])


@functools.cache
def _count_kernel():
    return pl.kernel(
        _count_body,
        out_type=[
            jax.ShapeDtypeStruct((VOCAB,), jnp.float32),
            jax.ShapeDtypeStruct((VOCAB,), jnp.float32),
        ],
        mesh=plsc.VectorSubcoreMesh(
            core_axis_name="c", subcore_axis_name="s",
            num_cores=NUM_CORES, num_subcores=NUM_SUBCORES,
        ),
        scratch_types=[
            pltpu.VMEM((ROWS_PER_TILE, CB), jnp.int32),
            pltpu.VMEM((CB,), jnp.float32),
            pltpu.VMEM((CHUNK,), jnp.float32),
            pltpu.VMEM_SHARED((VOCAB,), jnp.float32),
            pltpu.SemaphoreType.DMA,
        ],
        compiler_params=pltpu.CompilerParams(use_tc_tiling_on_sc=False),
    )


BV = 25600
NB = (VOCAB + BV - 1) // BV


def _fused_body(c0_ref, c1_ref, et_ref, wt_ref, b_ref, o_ref, acc_ref):
    p = pl.program_id(0)
    j = pl.program_id(1)

    @pl.when(jnp.logical_and(p == 0, j == 0))
    def _():
        acc_ref[...] = jnp.zeros_like(acc_ref)

    @pl.when(p == 0)
    def _():
        col = j * BV + lax.broadcasted_iota(jnp.int32, (1, BV), 1)
        valid = col < VOCAB
        c = jnp.where(valid, (c0_ref[...] + c1_ref[...])[None, :], 0.0)
        et = jnp.where(jnp.broadcast_to(valid, (EMBED, BV)), et_ref[...], 0.0)
        acc_ref[...] += lax.dot_general(
            c, et, (((1,), (1,)), ((), ())),
            preferred_element_type=jnp.float32,
        )

    @pl.when(p == 1)
    def _():
        res = lax.dot_general(
            acc_ref[...], wt_ref[...], (((1,), (0,)), ((), ())),
            preferred_element_type=jnp.float32,
        )
        o_ref[...] = res[0] + b_ref[...]


def _fused(c0, c1, embT, WT, b):
    return pl.pallas_call(
        _fused_body,
        grid=(2, NB),
        in_specs=[
            pl.BlockSpec((BV,), lambda p, j: (j * (1 - p) + (NB - 1) * p,)),
            pl.BlockSpec((BV,), lambda p, j: (j * (1 - p) + (NB - 1) * p,)),
            pl.BlockSpec((EMBED, BV),
                         lambda p, j: (0, j * (1 - p) + (NB - 1) * p)),
            pl.BlockSpec((EMBED, BV), lambda p, j: (0, j * p)),
            pl.BlockSpec((BV,), lambda p, j: (j * p,)),
        ],
        out_specs=pl.BlockSpec((BV,), lambda p, j: (j * p,)),
        out_shape=jax.ShapeDtypeStruct((VOCAB,), jnp.float32),
        scratch_shapes=[pltpu.VMEM((1, EMBED), jnp.float32)],
    )(c0, c1, embT, WT, b)


def kernel(inputs, emb, W, b):
    idx2 = inputs.astype(jnp.int32).reshape(NROWS, CB)
    c0, c1 = _count_kernel()(idx2)
    return _fused(c0, c1, emb.T, W.T, b)
